# Initial kernel scaffold; baseline (speedup 1.0000x reference)
import jax
import jax.numpy as jnp
from jax.experimental import pallas as pl


def kernel(token_ids, weight):
    raise NotImplementedError("write your pallas kernel here")


# speedup vs baseline: 1.6857x; 1.6857x over previous
import functools

import jax
import jax.numpy as jnp
from jax import lax
from jax.experimental import pallas as pl
from jax.experimental.pallas import tpu as pltpu
from jax.experimental.pallas import tpu_sc as plsc

_D = 64
_CHUNK = 128
_NW = 32


@functools.partial(jax.jit, static_argnames=())
def _embed_lookup(flat_ids, weight):
    B = flat_ids.shape[0]
    assert B % (_NW * _CHUNK) == 0
    n_chunks = B // (_NW * _CHUNK)
    idx2d = flat_ids.reshape(B // _CHUNK, _CHUNK)

    mesh = plsc.VectorSubcoreMesh(core_axis_name="c", subcore_axis_name="s")

    @functools.partial(
        pl.kernel,
        out_type=jax.ShapeDtypeStruct((B, _D), jnp.float32),
        mesh=mesh,
        scratch_types=[
            pltpu.VMEM((n_chunks, _CHUNK), jnp.int32),
            pltpu.VMEM((_CHUNK, _D), jnp.float32),
            pltpu.SemaphoreType.DMA,
        ],
        compiler_params=pltpu.CompilerParams(use_tc_tiling_on_sc=False),
    )
    def body(idx_hbm, table_hbm, out_hbm, idx_v, rows_v, sem):
        wid = lax.axis_index("s") * 2 + lax.axis_index("c")
        row_base = wid * n_chunks
        out_base = wid * n_chunks * _CHUNK
        pltpu.sync_copy(idx_hbm.at[pl.ds(row_base, n_chunks)], idx_v)

        def step(j, carry):
            pltpu.async_copy(table_hbm.at[idx_v.at[j]], rows_v, sem).wait()
            pltpu.sync_copy(
                rows_v, out_hbm.at[pl.ds(out_base + j * _CHUNK, _CHUNK)]
            )
            return carry

        lax.fori_loop(0, n_chunks, step, 0)

    return body(idx2d, weight)


def kernel(token_ids, weight):
    B0, B1 = token_ids.shape
    flat = token_ids.reshape(-1).astype(jnp.int32)
    out = _embed_lookup(flat, weight)
    return out.reshape(B0, B1, _D)


# speedup vs baseline: 1.8707x; 1.1098x over previous
import functools

import jax
import jax.numpy as jnp
from jax import lax
from jax.experimental import pallas as pl
from jax.experimental.pallas import tpu as pltpu
from jax.experimental.pallas import tpu_sc as plsc

_D = 64
_CHUNK = 128
_NBUF = 4
_NW = 32


def _embed_lookup(flat_ids, weight):
    B = flat_ids.shape[0]
    assert B % (_NW * _CHUNK * _NBUF) == 0
    n_chunks = B // (_NW * _CHUNK)
    n_outer = n_chunks // _NBUF
    idx2d = flat_ids.reshape(B // _CHUNK, _CHUNK)

    mesh = plsc.VectorSubcoreMesh(core_axis_name="c", subcore_axis_name="s")

    @functools.partial(
        pl.kernel,
        out_type=jax.ShapeDtypeStruct((B, _D), jnp.float32),
        mesh=mesh,
        scratch_types=[
            pltpu.VMEM((n_chunks, _CHUNK), jnp.int32),
            pltpu.VMEM((_NBUF, _CHUNK, _D), jnp.float32),
            pltpu.SemaphoreType.DMA((_NBUF,)),
            pltpu.SemaphoreType.DMA((_NBUF,)),
        ],
        compiler_params=pltpu.CompilerParams(use_tc_tiling_on_sc=False),
    )
    def body(idx_hbm, table_hbm, out_hbm, idx_v, rows_v, sem_g, sem_w):
        wid = lax.axis_index("s") * 2 + lax.axis_index("c")
        row_base = wid * n_chunks
        out_base = wid * n_chunks * _CHUNK
        pltpu.sync_copy(idx_hbm.at[pl.ds(row_base, n_chunks)], idx_v)

        def start_gather(j, b):
            pltpu.async_copy(table_hbm.at[idx_v.at[j]], rows_v.at[b],
                             sem_g.at[b])

        def wait_gather(j, b):
            pltpu.make_async_copy(table_hbm.at[idx_v.at[j]], rows_v.at[b],
                                  sem_g.at[b]).wait()

        def start_write(j, b):
            pltpu.async_copy(
                rows_v.at[b],
                out_hbm.at[pl.ds(out_base + j * _CHUNK, _CHUNK)],
                sem_w.at[b])

        def wait_write(b):
            pltpu.make_async_copy(
                rows_v.at[b], out_hbm.at[pl.ds(out_base, _CHUNK)],
                sem_w.at[b]).wait()

        for b in range(_NBUF):
            start_gather(b, b)

        def outer(g, carry):
            for b in range(_NBUF):
                j = g * _NBUF + b
                wait_gather(j, b)
                start_write(j, b)
            for b in range(_NBUF):
                jn = (g + 1) * _NBUF + b
                wait_write(b)
                start_gather(jn, b)
            return carry

        lax.fori_loop(0, n_outer - 1, outer, 0)

        gl = n_outer - 1
        for b in range(_NBUF):
            j = gl * _NBUF + b
            wait_gather(j, b)
            start_write(j, b)
        for b in range(_NBUF):
            wait_write(b)

    return body(idx2d, weight)


def kernel(token_ids, weight):
    B0, B1 = token_ids.shape
    flat = token_ids.reshape(-1).astype(jnp.int32)
    out = _embed_lookup(flat, weight)
    return out.reshape(B0, B1, _D)
